# baseline (speedup 1.0000x reference)
import functools

import jax
import jax.numpy as jnp
from jax import lax
from jax.experimental import pallas as pl
from jax.experimental.pallas import tpu as pltpu
from jax.experimental.pallas import tpu_sc as plsc

_NC = 2
_NS = 16
_NW = _NC * _NS

_ROWS = 8
_NSETS = 3


def kernel(x, pe_table):
    B, L, D = x.shape
    rows_w = L // _NW
    n_chunks = rows_w // _ROWS

    mesh = plsc.VectorSubcoreMesh(
        core_axis_name="c", subcore_axis_name="s",
        num_cores=_NC, num_subcores=_NS)

    @functools.partial(
        pl.kernel,
        out_type=jax.ShapeDtypeStruct((B * L, D), jnp.float32),
        mesh=mesh,
        scratch_types=(
            [pltpu.VMEM((B * _ROWS, D), jnp.float32)
             for _ in range(_NSETS)]
            + [pltpu.VMEM((_ROWS, D), jnp.float32)
               for _ in range(_NSETS)]
            + [pltpu.SemaphoreType.DMA for _ in range(_NSETS)]
            + [pltpu.SemaphoreType.DMA for _ in range(_NSETS)]
            + [pltpu.SemaphoreType.DMA for _ in range(_NSETS)]
        ),
    )
    def run(x_hbm, pe_hbm, out_hbm, *scratch):
        bufx = scratch[:_NSETS]
        bufp = scratch[_NSETS:2 * _NSETS]
        ld_sem = scratch[2 * _NSETS:3 * _NSETS]
        st_sem = scratch[3 * _NSETS:4 * _NSETS]
        pe_sem = scratch[4 * _NSETS:]

        wid = lax.axis_index("s") * _NC + lax.axis_index("c")
        row_base = wid * rows_w

        def pe_load(c, s):
            return pltpu.make_async_copy(
                pe_hbm.at[pl.ds(row_base + c * _ROWS, _ROWS)],
                bufp[s], pe_sem[s])

        def x_copies(c, s):
            return [pltpu.make_async_copy(
                x_hbm.at[pl.ds(b * L + row_base + c * _ROWS, _ROWS)],
                bufx[s].at[pl.ds(b * _ROWS, _ROWS)], ld_sem[s])
                for b in range(B)]

        def out_copies(c, s):
            return [pltpu.make_async_copy(
                bufx[s].at[pl.ds(b * _ROWS, _ROWS)],
                out_hbm.at[pl.ds(b * L + row_base + c * _ROWS, _ROWS)],
                st_sem[s])
                for b in range(B)]

        def start(copies):
            for cp in copies:
                cp.start()

        def wait(copies):
            for cp in copies:
                cp.wait()

        def compute(s):
            bx, bp = bufx[s], bufp[s]

            @plsc.parallel_loop(0, _ROWS * D, step=16, unroll=8)
            def _(o):
                r = o // D
                cc = o % D
                v = bp[r, pl.ds(cc, 16)]
                for b in range(B):
                    plsc.addupdate(bx.at[b * _ROWS + r, pl.ds(cc, 16)], v)

        pe_load(0, 0).start()
        start(x_copies(0, 0))
        start(x_copies(jnp.int32(1), 1))
        pe_load(0, 0).wait()
        pe_load(jnp.int32(1), 1).start()
        wait(x_copies(0, 0))
        compute(0)
        start(out_copies(0, 0))
        start(x_copies(jnp.int32(2), 2))

        def body(j, carry):
            for k in range(3):
                s = (1 + k) % _NSETS
                c = 1 + 3 * j + k
                pe_load(c, s).wait()
                pe_load(c + 1, (s + 1) % _NSETS).start()
                wait(x_copies(c, s))
                compute(s)
                start(out_copies(c, s))

                @pl.when(c + 2 < n_chunks)
                def _():
                    wait(out_copies(c - 1, (s + 2) % _NSETS))
                    start(x_copies(c + 2, (s + 2) % _NSETS))
            return carry

        lax.fori_loop(0, (n_chunks - 2) // 3, body, 0)

        cl = n_chunks - 1
        sl = cl % _NSETS
        pe_load(jnp.int32(cl), sl).wait()
        wait(x_copies(jnp.int32(cl), sl))
        compute(sl)
        start(out_copies(jnp.int32(cl), sl))
        for c in range(n_chunks - 3, n_chunks):
            wait(out_copies(jnp.int32(c), c % _NSETS))

    out = run(x.reshape(B * L, D), pe_table[:L])
    return out.reshape(B, L, D)


# speedup vs baseline: 1.0021x; 1.0021x over previous
import functools

import jax
import jax.numpy as jnp
from jax import lax
from jax.experimental import pallas as pl
from jax.experimental.pallas import tpu as pltpu
from jax.experimental.pallas import tpu_sc as plsc

_NC = 2
_NS = 16
_NW = _NC * _NS

_ROWS = 8
_NSETS = 3


def kernel(x, pe_table):
    B, L, D = x.shape
    rows_w = L // _NW
    n_chunks = rows_w // _ROWS

    mesh = plsc.VectorSubcoreMesh(
        core_axis_name="c", subcore_axis_name="s",
        num_cores=_NC, num_subcores=_NS)

    @functools.partial(
        pl.kernel,
        out_type=jax.ShapeDtypeStruct((B * L, D), jnp.float32),
        mesh=mesh,
        scratch_types=(
            [pltpu.VMEM((B * _ROWS, D), jnp.float32)
             for _ in range(_NSETS)]
            + [pltpu.VMEM((_ROWS, D), jnp.float32)
               for _ in range(_NSETS)]
            + [pltpu.SemaphoreType.DMA for _ in range(_NSETS)]
            + [pltpu.SemaphoreType.DMA for _ in range(_NSETS)]
            + [pltpu.SemaphoreType.DMA for _ in range(_NSETS)]
        ),
    )
    def run(x_hbm, pe_hbm, out_hbm, *scratch):
        bufx = scratch[:_NSETS]
        bufp = scratch[_NSETS:2 * _NSETS]
        ld_sem = scratch[2 * _NSETS:3 * _NSETS]
        st_sem = scratch[3 * _NSETS:4 * _NSETS]
        pe_sem = scratch[4 * _NSETS:]

        wid = lax.axis_index("s") * _NC + lax.axis_index("c")
        row_base = wid * rows_w

        def pe_load(c, s):
            return pltpu.make_async_copy(
                pe_hbm.at[pl.ds(row_base + c * _ROWS, _ROWS)],
                bufp[s], pe_sem[s])

        def x_copies(c, s):
            return [pltpu.make_async_copy(
                x_hbm.at[pl.ds(b * L + row_base + c * _ROWS, _ROWS)],
                bufx[s].at[pl.ds(b * _ROWS, _ROWS)], ld_sem[s])
                for b in range(B)]

        def out_copies(c, s):
            return [pltpu.make_async_copy(
                bufx[s].at[pl.ds(b * _ROWS, _ROWS)],
                out_hbm.at[pl.ds(b * L + row_base + c * _ROWS, _ROWS)],
                st_sem[s])
                for b in range(B)]

        def start(copies):
            for cp in copies:
                cp.start()

        def wait(copies):
            for cp in copies:
                cp.wait()

        def compute(s):
            bx, bp = bufx[s], bufp[s]

            @plsc.parallel_loop(0, _ROWS * D, step=16, unroll=4)
            def _(o):
                r = o // D
                cc = o % D
                v = bp[r, pl.ds(cc, 16)]
                for b in range(B):
                    plsc.addupdate(bx.at[b * _ROWS + r, pl.ds(cc, 16)], v)

        pe_load(0, 0).start()
        start(x_copies(0, 0))
        start(x_copies(jnp.int32(1), 1))
        pe_load(0, 0).wait()
        pe_load(jnp.int32(1), 1).start()
        wait(x_copies(0, 0))
        compute(0)
        start(out_copies(0, 0))
        start(x_copies(jnp.int32(2), 2))

        def body(j, carry):
            for k in range(3):
                s = (1 + k) % _NSETS
                c = 1 + 3 * j + k
                pe_load(c, s).wait()
                pe_load(c + 1, (s + 1) % _NSETS).start()
                wait(x_copies(c, s))
                compute(s)
                start(out_copies(c, s))

                @pl.when(c + 2 < n_chunks)
                def _():
                    wait(out_copies(c - 1, (s + 2) % _NSETS))
                    start(x_copies(c + 2, (s + 2) % _NSETS))
            return carry

        lax.fori_loop(0, (n_chunks - 2) // 3, body, 0)

        cl = n_chunks - 1
        sl = cl % _NSETS
        pe_load(jnp.int32(cl), sl).wait()
        wait(x_copies(jnp.int32(cl), sl))
        compute(sl)
        start(out_copies(jnp.int32(cl), sl))
        for c in range(n_chunks - 3, n_chunks):
            wait(out_copies(jnp.int32(c), c % _NSETS))

    out = run(x.reshape(B * L, D), pe_table[:L])
    return out.reshape(B, L, D)
